# baseline (speedup 1.0000x reference)
import functools

import jax
import jax.numpy as jnp
from jax import lax
from jax.experimental import pallas as pl
from jax.experimental.pallas import tpu as pltpu
from jax.experimental.pallas import tpu_sc as plsc

N_ROWS = 16384
D = 128
NUM_SLICES = 8
NUM_KEYS = 2 * NUM_SLICES
PENALTY_WEIGHT = 0.1

SC_ROWS = N_ROWS // 2
NC = 2
NS = 16
NW = NC * NS
RPW = SC_ROWS // NW
IDX_CHUNK = 128
NCHUNK = RPW // IDX_CHUNK

_mesh = plsc.VectorSubcoreMesh(
    core_axis_name="c", subcore_axis_name="s", num_cores=NC, num_subcores=NS
)


@functools.partial(
    pl.kernel,
    out_type=jax.ShapeDtypeStruct((NC, NUM_KEYS, D), jnp.float32),
    mesh=_mesh,
    scratch_types=[
        pltpu.VMEM((NCHUNK, IDX_CHUNK, D), jnp.float32),
        pltpu.VMEM((RPW,), jnp.int32),
        pltpu.VMEM((RPW,), jnp.int32),
        pltpu.VMEM((NCHUNK, IDX_CHUNK), jnp.int32),
        pltpu.VMEM((NUM_KEYS, D), jnp.float32),
        pltpu.VMEM_SHARED((NUM_KEYS, D), jnp.float32),
        pltpu.SemaphoreType.DMA,
        pltpu.SemaphoreType.DMA,
    ],
)
def _segment_sums(mo_hbm, tg_hbm, ts_hbm, sums_out,
                  rows_v, tg_v, ts_v, keys_v, z_sums, acc_sums, sem0, sem1):
    cid = lax.axis_index("c")
    sid = lax.axis_index("s")
    wid = cid * NS + sid
    base = wid * RPW

    sems = (sem0, sem1)
    copies = []
    for j in range(NCHUNK):
        copies.append(pltpu.async_copy(
            mo_hbm.at[pl.ds(base + j * IDX_CHUNK, IDX_CHUNK)],
            rows_v.at[j], sems[j]))

    pltpu.sync_copy(tg_hbm.at[pl.ds(base, RPW)], tg_v)
    pltpu.sync_copy(ts_hbm.at[pl.ds(base, RPW)], ts_v)

    zero16 = jnp.zeros((16,), jnp.float32)
    for r in range(NUM_KEYS):
        for c in range(D // 16):
            z_sums[r, pl.ds(c * 16, 16)] = zero16

    for i in range(RPW // 16):
        k16 = tg_v[pl.ds(i * 16, 16)] * NUM_SLICES + ts_v[pl.ds(i * 16, 16)]
        keys_v[i // (IDX_CHUNK // 16), pl.ds((i % (IDX_CHUNK // 16)) * 16, 16)] = k16

    @pl.when(sid == 0)
    def _():
        pltpu.sync_copy(z_sums, acc_sums)

    plsc.subcore_barrier()

    for j in range(NCHUNK):
        copies[j].wait()
        pltpu.sync_copy(rows_v.at[j], acc_sums.at[keys_v.at[j]], add=True)

    plsc.subcore_barrier()

    @pl.when(sid == 0)
    def _():
        pltpu.sync_copy(acc_sums, sums_out.at[cid])


def _tc_partial_body(mo_ref, tg_ref, ts_ref, out_ref):
    key = tg_ref[...] * NUM_SLICES + ts_ref[...]
    onehot = (key == lax.broadcasted_iota(jnp.int32, (1, NUM_KEYS), 1))
    onehot = onehot.astype(jnp.float32)
    out_ref[...] = lax.dot_general(
        onehot, mo_ref[...], (((0,), (0,)), ((), ())),
        preferred_element_type=jnp.float32)


_tc_partial = pl.pallas_call(
    _tc_partial_body,
    out_shape=jax.ShapeDtypeStruct((NUM_KEYS, D), jnp.float32),
)


def _finalize_body(sc_ref, tc_ref, tg_ref, ts_ref, out_ref):
    s = sc_ref[...]
    s16 = s[0] + s[1] + tc_ref[...]
    key = tg_ref[...] * NUM_SLICES + ts_ref[...]

    cnt = []
    for b in range(NUM_KEYS):
        cnt.append(jnp.sum((key == b).astype(jnp.float32)))

    n = jnp.float32(0.0)
    present = []
    for t in range(NUM_SLICES):
        p = jnp.where(cnt[t] + cnt[NUM_SLICES + t] > 0, 1.0, 0.0)
        present.append(p)
        n = n + p

    inv_n = 1.0 / jnp.maximum(n, 1.0)
    inv_nm1 = 1.0 / jnp.maximum(n - 1.0, 1.0)

    def var_mean(base_key):
        means = []
        for t in range(NUM_SLICES):
            c = cnt[base_key + t]
            inv_c = jnp.where(c > 0, 1.0 / jnp.maximum(c, 1.0), 0.0)
            means.append(s16[base_key + t:base_key + t + 1] * inv_c)
        mu = means[0] * present[0]
        for t in range(1, NUM_SLICES):
            mu = mu + means[t] * present[t]
        mu = mu * inv_n
        var = ((means[0] - mu) ** 2) * present[0]
        for t in range(1, NUM_SLICES):
            var = var + ((means[t] - mu) ** 2) * present[t]
        var = var * inv_nm1
        return jnp.mean(var, axis=1, keepdims=True)

    penalty = (var_mean(0) + var_mean(NUM_SLICES)) / 2.0
    penalty = jnp.where(n < 2, 0.0, PENALTY_WEIGHT * penalty)
    out_ref[...] = penalty


_finalize = pl.pallas_call(
    _finalize_body,
    out_shape=jax.ShapeDtypeStruct((1, 1), jnp.float32),
)


def kernel(model_output, targets, time_slices):
    tg = targets.astype(jnp.int32)
    ts = time_slices.astype(jnp.int32)
    sc_sums = _segment_sums(model_output[:SC_ROWS], tg[:SC_ROWS], ts[:SC_ROWS])
    tc_sums = _tc_partial(model_output[SC_ROWS:],
                          tg[SC_ROWS:].reshape(N_ROWS - SC_ROWS, 1),
                          ts[SC_ROWS:].reshape(N_ROWS - SC_ROWS, 1))
    out = _finalize(sc_sums, tc_sums,
                    tg.reshape(N_ROWS // D, D),
                    ts.reshape(N_ROWS // D, D))
    return out[0, 0]


# speedup vs baseline: 1.2087x; 1.2087x over previous
import functools

import jax
import jax.numpy as jnp
from jax import lax
from jax.experimental import pallas as pl
from jax.experimental.pallas import tpu as pltpu
from jax.experimental.pallas import tpu_sc as plsc

N_ROWS = 16384
D = 128
NUM_SLICES = 8
NUM_KEYS = 2 * NUM_SLICES
PENALTY_WEIGHT = 0.1

SC_ROWS = N_ROWS // 2
NC = 2
NS = 16
NW = NC * NS
RPW = SC_ROWS // NW
IDX_CHUNK = 128
NCHUNK = RPW // IDX_CHUNK

_mesh = plsc.VectorSubcoreMesh(
    core_axis_name="c", subcore_axis_name="s", num_cores=NC, num_subcores=NS
)


@functools.partial(
    pl.kernel,
    out_type=jax.ShapeDtypeStruct((NC, NUM_KEYS, D), jnp.float32),
    mesh=_mesh,
    scratch_types=[
        pltpu.VMEM((NCHUNK, IDX_CHUNK, D), jnp.float32),
        pltpu.VMEM((RPW,), jnp.int32),
        pltpu.VMEM((RPW,), jnp.int32),
        pltpu.VMEM((NCHUNK, IDX_CHUNK), jnp.int32),
        pltpu.VMEM((NUM_KEYS, D), jnp.float32),
        pltpu.VMEM_SHARED((NUM_KEYS, D), jnp.float32),
        pltpu.SemaphoreType.DMA,
        pltpu.SemaphoreType.DMA,
    ],
)
def _segment_sums(mo_hbm, tg_hbm, ts_hbm, sums_out,
                  rows_v, tg_v, ts_v, keys_v, z_sums, acc_sums, sem0, sem1):
    cid = lax.axis_index("c")
    sid = lax.axis_index("s")
    wid = cid * NS + sid
    base = wid * RPW

    sems = (sem0, sem1)
    copies = []
    for j in range(NCHUNK):
        copies.append(pltpu.async_copy(
            mo_hbm.at[pl.ds(base + j * IDX_CHUNK, IDX_CHUNK)],
            rows_v.at[j], sems[j]))

    pltpu.sync_copy(tg_hbm.at[pl.ds(base, RPW)], tg_v)
    pltpu.sync_copy(ts_hbm.at[pl.ds(base, RPW)], ts_v)

    zero16 = jnp.zeros((16,), jnp.float32)
    for r in range(NUM_KEYS):
        for c in range(D // 16):
            z_sums[r, pl.ds(c * 16, 16)] = zero16

    for i in range(RPW // 16):
        k16 = tg_v[pl.ds(i * 16, 16)] * NUM_SLICES + ts_v[pl.ds(i * 16, 16)]
        keys_v[i // (IDX_CHUNK // 16), pl.ds((i % (IDX_CHUNK // 16)) * 16, 16)] = k16

    @pl.when(sid == 0)
    def _():
        pltpu.sync_copy(z_sums, acc_sums)

    plsc.subcore_barrier()

    for j in range(NCHUNK):
        copies[j].wait()
        pltpu.sync_copy(rows_v.at[j], acc_sums.at[keys_v.at[j]], add=True)

    plsc.subcore_barrier()

    @pl.when(sid == 0)
    def _():
        pltpu.sync_copy(acc_sums, sums_out.at[cid])


def _tc_partial_body(mo_ref, tg_ref, ts_ref, out_ref):
    key = tg_ref[...] * NUM_SLICES + ts_ref[...]
    onehot_t = (key == lax.broadcasted_iota(jnp.int32, (NUM_KEYS, 1), 0))
    onehot_t = onehot_t.astype(jnp.float32)
    out_ref[...] = lax.dot_general(
        onehot_t, mo_ref[...], (((1,), (0,)), ((), ())),
        preferred_element_type=jnp.float32)


_tc_partial = pl.pallas_call(
    _tc_partial_body,
    out_shape=jax.ShapeDtypeStruct((NUM_KEYS, D), jnp.float32),
)


def _finalize_body(sc_ref, tc_ref, tg_ref, ts_ref, out_ref):
    s = sc_ref[...]
    s16 = s[0] + s[1] + tc_ref[...]
    key = tg_ref[...] * NUM_SLICES + ts_ref[...]

    cnt = []
    for b in range(NUM_KEYS):
        cnt.append(jnp.sum((key == b).astype(jnp.float32)))

    n = jnp.float32(0.0)
    present = []
    for t in range(NUM_SLICES):
        p = jnp.where(cnt[t] + cnt[NUM_SLICES + t] > 0, 1.0, 0.0)
        present.append(p)
        n = n + p

    inv_n = 1.0 / jnp.maximum(n, 1.0)
    inv_nm1 = 1.0 / jnp.maximum(n - 1.0, 1.0)

    def var_mean(base_key):
        means = []
        for t in range(NUM_SLICES):
            c = cnt[base_key + t]
            inv_c = jnp.where(c > 0, 1.0 / jnp.maximum(c, 1.0), 0.0)
            means.append(s16[base_key + t:base_key + t + 1] * inv_c)
        mu = means[0] * present[0]
        for t in range(1, NUM_SLICES):
            mu = mu + means[t] * present[t]
        mu = mu * inv_n
        var = ((means[0] - mu) ** 2) * present[0]
        for t in range(1, NUM_SLICES):
            var = var + ((means[t] - mu) ** 2) * present[t]
        var = var * inv_nm1
        return jnp.mean(var, axis=1, keepdims=True)

    penalty = (var_mean(0) + var_mean(NUM_SLICES)) / 2.0
    penalty = jnp.where(n < 2, 0.0, PENALTY_WEIGHT * penalty)
    out_ref[...] = penalty


_finalize = pl.pallas_call(
    _finalize_body,
    out_shape=jax.ShapeDtypeStruct((1, 1), jnp.float32),
)


def kernel(model_output, targets, time_slices):
    tg = targets.astype(jnp.int32)
    ts = time_slices.astype(jnp.int32)
    sc_sums = _segment_sums(model_output[:SC_ROWS], tg[:SC_ROWS], ts[:SC_ROWS])
    tc_sums = _tc_partial(model_output[SC_ROWS:],
                          tg[SC_ROWS:].reshape(1, N_ROWS - SC_ROWS),
                          ts[SC_ROWS:].reshape(1, N_ROWS - SC_ROWS))
    out = _finalize(sc_sums, tc_sums,
                    tg.reshape(N_ROWS // D, D),
                    ts.reshape(N_ROWS // D, D))
    return out[0, 0]


# speedup vs baseline: 1.4760x; 1.2212x over previous
import functools

import jax
import jax.numpy as jnp
from jax import lax
from jax.experimental import pallas as pl
from jax.experimental.pallas import tpu as pltpu
from jax.experimental.pallas import tpu_sc as plsc

N_ROWS = 16384
D = 128
NUM_SLICES = 8
NUM_KEYS = 2 * NUM_SLICES
PENALTY_WEIGHT = 0.1

SC_ROWS = N_ROWS // 2
NC = 2
NS = 16
NW = NC * NS
RPW = SC_ROWS // NW
IDX_CHUNK = 128
NCHUNK = RPW // IDX_CHUNK

_mesh = plsc.VectorSubcoreMesh(
    core_axis_name="c", subcore_axis_name="s", num_cores=NC, num_subcores=NS
)


@functools.partial(
    pl.kernel,
    out_type=jax.ShapeDtypeStruct((NC, NUM_KEYS, D), jnp.float32),
    mesh=_mesh,
    scratch_types=[
        pltpu.VMEM((NCHUNK, IDX_CHUNK, D), jnp.float32),
        pltpu.VMEM((RPW,), jnp.int32),
        pltpu.VMEM((RPW,), jnp.int32),
        pltpu.VMEM((NCHUNK, IDX_CHUNK), jnp.int32),
        pltpu.VMEM((NUM_KEYS, D), jnp.float32),
        pltpu.VMEM_SHARED((NUM_KEYS, D), jnp.float32),
        pltpu.SemaphoreType.DMA,
        pltpu.SemaphoreType.DMA,
    ],
)
def _segment_sums(mo_hbm, tg_hbm, ts_hbm, sums_out,
                  rows_v, tg_v, ts_v, keys_v, z_sums, acc_sums, sem0, sem1):
    cid = lax.axis_index("c")
    sid = lax.axis_index("s")
    wid = cid * NS + sid
    base = wid * RPW

    sems = (sem0, sem1)
    copies = []
    for j in range(NCHUNK):
        copies.append(pltpu.async_copy(
            mo_hbm.at[pl.ds(base + j * IDX_CHUNK, IDX_CHUNK)],
            rows_v.at[j], sems[j]))

    pltpu.sync_copy(tg_hbm.at[pl.ds(base, RPW)], tg_v)
    pltpu.sync_copy(ts_hbm.at[pl.ds(base, RPW)], ts_v)

    zero16 = jnp.zeros((16,), jnp.float32)
    for r in range(NUM_KEYS):
        for c in range(D // 16):
            z_sums[r, pl.ds(c * 16, 16)] = zero16

    for i in range(RPW // 16):
        k16 = tg_v[pl.ds(i * 16, 16)] * NUM_SLICES + ts_v[pl.ds(i * 16, 16)]
        keys_v[i // (IDX_CHUNK // 16), pl.ds((i % (IDX_CHUNK // 16)) * 16, 16)] = k16

    @pl.when(sid == 0)
    def _():
        pltpu.sync_copy(z_sums, acc_sums)

    plsc.subcore_barrier()

    for j in range(NCHUNK):
        copies[j].wait()
        pltpu.sync_copy(rows_v.at[j], acc_sums.at[keys_v.at[j]], add=True)

    plsc.subcore_barrier()

    @pl.when(sid == 0)
    def _():
        pltpu.sync_copy(acc_sums, sums_out.at[cid])


TC_BLOCK = 2048
TC_GRID = (N_ROWS - SC_ROWS) // TC_BLOCK


def _tc_partial_body(mo_ref, tg_ref, ts_ref, out_ref):
    i = pl.program_id(0)

    @pl.when(i == 0)
    def _():
        out_ref[...] = jnp.zeros((NUM_KEYS, D), jnp.float32)

    key = tg_ref[...] * NUM_SLICES + ts_ref[...]
    onehot_t = (key == lax.broadcasted_iota(jnp.int32, (NUM_KEYS, 1), 0))
    onehot_t = onehot_t.astype(jnp.float32)
    out_ref[...] += lax.dot_general(
        onehot_t, mo_ref[...], (((1,), (0,)), ((), ())),
        preferred_element_type=jnp.float32)


_tc_partial = pl.pallas_call(
    _tc_partial_body,
    grid=(TC_GRID,),
    in_specs=[
        pl.BlockSpec((TC_BLOCK, D), lambda i: (SC_ROWS // TC_BLOCK + i, 0)),
        pl.BlockSpec((1, TC_BLOCK), lambda i: (0, SC_ROWS // TC_BLOCK + i)),
        pl.BlockSpec((1, TC_BLOCK), lambda i: (0, SC_ROWS // TC_BLOCK + i)),
    ],
    out_specs=pl.BlockSpec((NUM_KEYS, D), lambda i: (0, 0)),
    out_shape=jax.ShapeDtypeStruct((NUM_KEYS, D), jnp.float32),
)


def _finalize_body(sc_ref, tc_ref, tg_ref, ts_ref, out_ref):
    s = sc_ref[...]
    s16 = s[0] + s[1] + tc_ref[...]
    key = tg_ref[...] * NUM_SLICES + ts_ref[...]

    cnt = []
    for b in range(NUM_KEYS):
        cnt.append(jnp.sum((key == b).astype(jnp.float32)))

    n = jnp.float32(0.0)
    present = []
    for t in range(NUM_SLICES):
        p = jnp.where(cnt[t] + cnt[NUM_SLICES + t] > 0, 1.0, 0.0)
        present.append(p)
        n = n + p

    inv_n = 1.0 / jnp.maximum(n, 1.0)
    inv_nm1 = 1.0 / jnp.maximum(n - 1.0, 1.0)

    def var_mean(base_key):
        means = []
        for t in range(NUM_SLICES):
            c = cnt[base_key + t]
            inv_c = jnp.where(c > 0, 1.0 / jnp.maximum(c, 1.0), 0.0)
            means.append(s16[base_key + t:base_key + t + 1] * inv_c)
        mu = means[0] * present[0]
        for t in range(1, NUM_SLICES):
            mu = mu + means[t] * present[t]
        mu = mu * inv_n
        var = ((means[0] - mu) ** 2) * present[0]
        for t in range(1, NUM_SLICES):
            var = var + ((means[t] - mu) ** 2) * present[t]
        var = var * inv_nm1
        return jnp.mean(var, axis=1, keepdims=True)

    penalty = (var_mean(0) + var_mean(NUM_SLICES)) / 2.0
    penalty = jnp.where(n < 2, 0.0, PENALTY_WEIGHT * penalty)
    out_ref[...] = penalty


_finalize = pl.pallas_call(
    _finalize_body,
    out_shape=jax.ShapeDtypeStruct((1, 1), jnp.float32),
)


def kernel(model_output, targets, time_slices):
    tg = targets.astype(jnp.int32)
    ts = time_slices.astype(jnp.int32)
    tg_row = tg.reshape(1, N_ROWS)
    ts_row = ts.reshape(1, N_ROWS)
    sc_sums = _segment_sums(model_output, tg, ts)
    tc_sums = _tc_partial(model_output, tg_row, ts_row)
    out = _finalize(sc_sums, tc_sums,
                    tg.reshape(N_ROWS // D, D),
                    ts.reshape(N_ROWS // D, D))
    return out[0, 0]


# speedup vs baseline: 1.5246x; 1.0329x over previous
import functools

import jax
import jax.numpy as jnp
from jax import lax
from jax.experimental import pallas as pl
from jax.experimental.pallas import tpu as pltpu
from jax.experimental.pallas import tpu_sc as plsc

N_ROWS = 16384
D = 128
NUM_SLICES = 8
NUM_KEYS = 2 * NUM_SLICES
PENALTY_WEIGHT = 0.1

SC_ROWS = N_ROWS // 4
NC = 2
NS = 16
NW = NC * NS
RPW = SC_ROWS // NW
IDX_CHUNK = 128
NCHUNK = RPW // IDX_CHUNK

_mesh = plsc.VectorSubcoreMesh(
    core_axis_name="c", subcore_axis_name="s", num_cores=NC, num_subcores=NS
)


@functools.partial(
    pl.kernel,
    out_type=jax.ShapeDtypeStruct((NC, NUM_KEYS, D), jnp.float32),
    mesh=_mesh,
    scratch_types=[
        pltpu.VMEM((NCHUNK, IDX_CHUNK, D), jnp.float32),
        pltpu.VMEM((RPW,), jnp.int32),
        pltpu.VMEM((RPW,), jnp.int32),
        pltpu.VMEM((NCHUNK, IDX_CHUNK), jnp.int32),
        pltpu.VMEM((NUM_KEYS, D), jnp.float32),
        pltpu.VMEM_SHARED((NUM_KEYS, D), jnp.float32),
        pltpu.SemaphoreType.DMA,
    ],
)
def _segment_sums(mo_hbm, tg_hbm, ts_hbm, sums_out,
                  rows_v, tg_v, ts_v, keys_v, z_sums, acc_sums, sem0):
    cid = lax.axis_index("c")
    sid = lax.axis_index("s")
    wid = cid * NS + sid
    base = wid * RPW

    copies = []
    for j in range(NCHUNK):
        copies.append(pltpu.async_copy(
            mo_hbm.at[pl.ds(base + j * IDX_CHUNK, IDX_CHUNK)],
            rows_v.at[j], sem0))

    pltpu.sync_copy(tg_hbm.at[pl.ds(base, RPW)], tg_v)
    pltpu.sync_copy(ts_hbm.at[pl.ds(base, RPW)], ts_v)

    zero16 = jnp.zeros((16,), jnp.float32)
    for r in range(NUM_KEYS):
        for c in range(D // 16):
            z_sums[r, pl.ds(c * 16, 16)] = zero16

    for i in range(RPW // 16):
        k16 = tg_v[pl.ds(i * 16, 16)] * NUM_SLICES + ts_v[pl.ds(i * 16, 16)]
        keys_v[i // (IDX_CHUNK // 16), pl.ds((i % (IDX_CHUNK // 16)) * 16, 16)] = k16

    @pl.when(sid == 0)
    def _():
        pltpu.sync_copy(z_sums, acc_sums)

    plsc.subcore_barrier()

    for j in range(NCHUNK):
        copies[j].wait()
        pltpu.sync_copy(rows_v.at[j], acc_sums.at[keys_v.at[j]], add=True)

    plsc.subcore_barrier()

    @pl.when(sid == 0)
    def _():
        pltpu.sync_copy(acc_sums, sums_out.at[cid])


TC_BLOCK = 2048
TC_GRID = (N_ROWS - SC_ROWS) // TC_BLOCK


def _tc_partial_body(mo_ref, tg_ref, ts_ref, out_ref):
    i = pl.program_id(0)

    @pl.when(i == 0)
    def _():
        out_ref[...] = jnp.zeros((NUM_KEYS, D), jnp.float32)

    key = tg_ref[...] * NUM_SLICES + ts_ref[...]
    onehot_t = (key == lax.broadcasted_iota(jnp.int32, (NUM_KEYS, 1), 0))
    onehot_t = onehot_t.astype(jnp.float32)
    out_ref[...] += lax.dot_general(
        onehot_t, mo_ref[...], (((1,), (0,)), ((), ())),
        preferred_element_type=jnp.float32)


_tc_partial = pl.pallas_call(
    _tc_partial_body,
    grid=(TC_GRID,),
    in_specs=[
        pl.BlockSpec((TC_BLOCK, D), lambda i: (SC_ROWS // TC_BLOCK + i, 0)),
        pl.BlockSpec((1, TC_BLOCK), lambda i: (0, SC_ROWS // TC_BLOCK + i)),
        pl.BlockSpec((1, TC_BLOCK), lambda i: (0, SC_ROWS // TC_BLOCK + i)),
    ],
    out_specs=pl.BlockSpec((NUM_KEYS, D), lambda i: (0, 0)),
    out_shape=jax.ShapeDtypeStruct((NUM_KEYS, D), jnp.float32),
)


def _finalize_body(sc_ref, tc_ref, tg_ref, ts_ref, out_ref):
    s = sc_ref[...]
    s16 = s[0] + s[1] + tc_ref[...]
    key = tg_ref[...] * NUM_SLICES + ts_ref[...]
    onehot_t = (key == lax.broadcasted_iota(jnp.int32, (NUM_KEYS, 1), 0))
    cnt = jnp.sum(onehot_t.astype(jnp.float32), axis=1, keepdims=True)

    neg_cnt, pos_cnt = cnt[:NUM_SLICES], cnt[NUM_SLICES:]
    present = (neg_cnt + pos_cnt > 0).astype(jnp.float32)
    n = jnp.sum(present, axis=(0, 1), keepdims=True)

    inv_n = 1.0 / jnp.maximum(n, 1.0)
    inv_nm1 = 1.0 / jnp.maximum(n - 1.0, 1.0)

    def var_mean(sums, c):
        means = jnp.where(c > 0, sums / jnp.maximum(c, 1.0), 0.0)
        mu = jnp.sum(means * present, axis=0, keepdims=True) * inv_n
        var = jnp.sum(((means - mu) ** 2) * present, axis=0,
                      keepdims=True) * inv_nm1
        return jnp.mean(var, axis=1, keepdims=True)

    penalty = (var_mean(s16[:NUM_SLICES], neg_cnt)
               + var_mean(s16[NUM_SLICES:], pos_cnt)) / 2.0
    penalty = jnp.where(n < 2, 0.0, PENALTY_WEIGHT * penalty)
    out_ref[...] = penalty


_finalize = pl.pallas_call(
    _finalize_body,
    out_shape=jax.ShapeDtypeStruct((1, 1), jnp.float32),
)


def kernel(model_output, targets, time_slices):
    tg = targets.astype(jnp.int32)
    ts = time_slices.astype(jnp.int32)
    tg_row = tg.reshape(1, N_ROWS)
    ts_row = ts.reshape(1, N_ROWS)
    sc_sums = _segment_sums(model_output, tg, ts)
    tc_sums = _tc_partial(model_output, tg_row, ts_row)
    out = _finalize(sc_sums, tc_sums, tg_row, ts_row)
    return out[0, 0]


# speedup vs baseline: 1.5518x; 1.0179x over previous
import functools

import jax
import jax.numpy as jnp
from jax import lax
from jax.experimental import pallas as pl
from jax.experimental.pallas import tpu as pltpu
from jax.experimental.pallas import tpu_sc as plsc

N_ROWS = 16384
D = 128
NUM_SLICES = 8
NUM_KEYS = 2 * NUM_SLICES
PENALTY_WEIGHT = 0.1

SC_ROWS = N_ROWS // 4
NC = 2
NS = 16
NW = NC * NS
RPW = SC_ROWS // NW
IDX_CHUNK = 128
NCHUNK = RPW // IDX_CHUNK

_mesh = plsc.VectorSubcoreMesh(
    core_axis_name="c", subcore_axis_name="s", num_cores=NC, num_subcores=NS
)


@functools.partial(
    pl.kernel,
    out_type=jax.ShapeDtypeStruct((NC, NUM_KEYS, D), jnp.float32),
    mesh=_mesh,
    scratch_types=[
        pltpu.VMEM((NCHUNK, IDX_CHUNK, D), jnp.float32),
        pltpu.VMEM((RPW,), jnp.int32),
        pltpu.VMEM((RPW,), jnp.int32),
        pltpu.VMEM((NCHUNK, IDX_CHUNK), jnp.int32),
        pltpu.VMEM((NUM_KEYS, D), jnp.float32),
        pltpu.VMEM_SHARED((NUM_KEYS, D), jnp.float32),
        pltpu.SemaphoreType.DMA,
    ],
)
def _segment_sums(mo_hbm, tg_hbm, ts_hbm, sums_out,
                  rows_v, tg_v, ts_v, keys_v, z_sums, acc_sums, sem0):
    cid = lax.axis_index("c")
    sid = lax.axis_index("s")
    wid = cid * NS + sid
    base = wid * RPW

    copies = []
    for j in range(NCHUNK):
        copies.append(pltpu.async_copy(
            mo_hbm.at[pl.ds(base + j * IDX_CHUNK, IDX_CHUNK)],
            rows_v.at[j], sem0))

    pltpu.sync_copy(tg_hbm.at[pl.ds(base, RPW)], tg_v)
    pltpu.sync_copy(ts_hbm.at[pl.ds(base, RPW)], ts_v)

    zero16 = jnp.zeros((16,), jnp.float32)

    def _zero_row(r, carry):
        for c in range(D // 16):
            z_sums[r, pl.ds(c * 16, 16)] = zero16
        return carry

    lax.fori_loop(0, NUM_KEYS, _zero_row, 0)

    for i in range(RPW // 16):
        k16 = tg_v[pl.ds(i * 16, 16)] * NUM_SLICES + ts_v[pl.ds(i * 16, 16)]
        keys_v[i // (IDX_CHUNK // 16), pl.ds((i % (IDX_CHUNK // 16)) * 16, 16)] = k16

    @pl.when(sid == 0)
    def _():
        pltpu.sync_copy(z_sums, acc_sums)

    plsc.subcore_barrier()

    for j in range(NCHUNK):
        copies[j].wait()
        pltpu.sync_copy(rows_v.at[j], acc_sums.at[keys_v.at[j]], add=True)

    plsc.subcore_barrier()

    @pl.when(sid == 0)
    def _():
        pltpu.sync_copy(acc_sums, sums_out.at[cid])


TC_BLOCK = 4096
TC_GRID = (N_ROWS - SC_ROWS) // TC_BLOCK


def _tc_partial_body(mo_ref, tg_ref, ts_ref, out_ref, cnt_ref):
    i = pl.program_id(0)

    @pl.when(i == 0)
    def _():
        out_ref[...] = jnp.zeros((NUM_KEYS, D), jnp.float32)
        cnt_ref[...] = jnp.zeros((NUM_KEYS, D), jnp.float32)

    key = tg_ref[...] * NUM_SLICES + ts_ref[...]
    onehot_t = (key == lax.broadcasted_iota(jnp.int32, (NUM_KEYS, 1), 0))
    onehot_t = onehot_t.astype(jnp.float32)
    out_ref[...] += lax.dot_general(
        onehot_t, mo_ref[...], (((1,), (0,)), ((), ())),
        preferred_element_type=jnp.float32)
    cnt_ref[...] += jnp.broadcast_to(
        jnp.sum(onehot_t, axis=1, keepdims=True), (NUM_KEYS, D))


_tc_partial = pl.pallas_call(
    _tc_partial_body,
    grid=(TC_GRID,),
    in_specs=[
        pl.BlockSpec((TC_BLOCK, D), lambda i: (SC_ROWS // TC_BLOCK + i, 0)),
        pl.BlockSpec((1, TC_BLOCK), lambda i: (0, SC_ROWS // TC_BLOCK + i)),
        pl.BlockSpec((1, TC_BLOCK), lambda i: (0, SC_ROWS // TC_BLOCK + i)),
    ],
    out_specs=(pl.BlockSpec((NUM_KEYS, D), lambda i: (0, 0)),
               pl.BlockSpec((NUM_KEYS, D), lambda i: (0, 0))),
    out_shape=(jax.ShapeDtypeStruct((NUM_KEYS, D), jnp.float32),
               jax.ShapeDtypeStruct((NUM_KEYS, D), jnp.float32)),
)


def _finalize_body(sc_ref, tc_ref, tc_cnt_ref, tg_ref, ts_ref, out_ref):
    s = sc_ref[...]
    s16 = s[0] + s[1] + tc_ref[...]
    key = tg_ref[...] * NUM_SLICES + ts_ref[...]
    onehot_t = (key == lax.broadcasted_iota(jnp.int32, (NUM_KEYS, 1), 0))
    cnt = jnp.sum(onehot_t.astype(jnp.float32), axis=1, keepdims=True)
    cnt = cnt + tc_cnt_ref[...][:, 0:1]

    neg_cnt, pos_cnt = cnt[:NUM_SLICES], cnt[NUM_SLICES:]
    present = (neg_cnt + pos_cnt > 0).astype(jnp.float32)
    n = jnp.sum(present, axis=(0, 1), keepdims=True)

    inv_n = 1.0 / jnp.maximum(n, 1.0)
    inv_nm1 = 1.0 / jnp.maximum(n - 1.0, 1.0)

    def var_mean(sums, c):
        means = jnp.where(c > 0, sums / jnp.maximum(c, 1.0), 0.0)
        mu = jnp.sum(means * present, axis=0, keepdims=True) * inv_n
        var = jnp.sum(((means - mu) ** 2) * present, axis=0,
                      keepdims=True) * inv_nm1
        return jnp.mean(var, axis=1, keepdims=True)

    penalty = (var_mean(s16[:NUM_SLICES], neg_cnt)
               + var_mean(s16[NUM_SLICES:], pos_cnt)) / 2.0
    penalty = jnp.where(n < 2, 0.0, PENALTY_WEIGHT * penalty)
    out_ref[...] = penalty


_finalize = pl.pallas_call(
    _finalize_body,
    grid=(1,),
    in_specs=[
        pl.BlockSpec((NC, NUM_KEYS, D), lambda i: (0, 0, 0)),
        pl.BlockSpec((NUM_KEYS, D), lambda i: (0, 0)),
        pl.BlockSpec((NUM_KEYS, D), lambda i: (0, 0)),
        pl.BlockSpec((1, SC_ROWS), lambda i: (0, 0)),
        pl.BlockSpec((1, SC_ROWS), lambda i: (0, 0)),
    ],
    out_specs=pl.BlockSpec((1, 1), lambda i: (0, 0)),
    out_shape=jax.ShapeDtypeStruct((1, 1), jnp.float32),
)


def kernel(model_output, targets, time_slices):
    tg = targets.astype(jnp.int32)
    ts = time_slices.astype(jnp.int32)
    tg_row = tg.reshape(1, N_ROWS)
    ts_row = ts.reshape(1, N_ROWS)
    sc_sums = _segment_sums(model_output, tg, ts)
    tc_sums, tc_cnts = _tc_partial(model_output, tg_row, ts_row)
    out = _finalize(sc_sums, tc_sums, tc_cnts, tg_row, ts_row)
    return out[0, 0]


# speedup vs baseline: 1.5605x; 1.0056x over previous
import functools

import jax
import jax.numpy as jnp
from jax import lax
from jax.experimental import pallas as pl
from jax.experimental.pallas import tpu as pltpu
from jax.experimental.pallas import tpu_sc as plsc

N_ROWS = 16384
D = 128
NUM_SLICES = 8
NUM_KEYS = 2 * NUM_SLICES
PENALTY_WEIGHT = 0.1

SC_ROWS = N_ROWS // 4
NC = 2
NS = 16
NW = NC * NS
RPW = SC_ROWS // NW
IDX_CHUNK = 128
NCHUNK = RPW // IDX_CHUNK

_mesh = plsc.VectorSubcoreMesh(
    core_axis_name="c", subcore_axis_name="s", num_cores=NC, num_subcores=NS
)


@functools.partial(
    pl.kernel,
    out_type=jax.ShapeDtypeStruct((NC, NUM_KEYS, D), jnp.float32),
    mesh=_mesh,
    scratch_types=[
        pltpu.VMEM((NCHUNK, IDX_CHUNK, D), jnp.float32),
        pltpu.VMEM((RPW,), jnp.int32),
        pltpu.VMEM((RPW,), jnp.int32),
        pltpu.VMEM((NCHUNK, IDX_CHUNK), jnp.int32),
        pltpu.VMEM((NUM_KEYS, D), jnp.float32),
        pltpu.VMEM_SHARED((NUM_KEYS, D), jnp.float32),
        pltpu.SemaphoreType.DMA,
    ],
)
def _segment_sums(mo_hbm, tg_hbm, ts_hbm, sums_out,
                  rows_v, tg_v, ts_v, keys_v, z_sums, acc_sums, sem0):
    cid = lax.axis_index("c")
    sid = lax.axis_index("s")
    wid = cid * NS + sid
    base = wid * RPW

    copies = []
    for j in range(NCHUNK):
        copies.append(pltpu.async_copy(
            mo_hbm.at[pl.ds(base + j * IDX_CHUNK, IDX_CHUNK)],
            rows_v.at[j], sem0))

    pltpu.sync_copy(tg_hbm.at[pl.ds(base, RPW)], tg_v)
    pltpu.sync_copy(ts_hbm.at[pl.ds(base, RPW)], ts_v)

    zero16 = jnp.zeros((16,), jnp.float32)

    def _zero_row(r, carry):
        for c in range(D // 16):
            z_sums[r, pl.ds(c * 16, 16)] = zero16
        return carry

    lax.fori_loop(0, NUM_KEYS, _zero_row, 0)

    for i in range(RPW // 16):
        k16 = tg_v[pl.ds(i * 16, 16)] * NUM_SLICES + ts_v[pl.ds(i * 16, 16)]
        keys_v[i // (IDX_CHUNK // 16), pl.ds((i % (IDX_CHUNK // 16)) * 16, 16)] = k16

    @pl.when(sid == 0)
    def _():
        pltpu.sync_copy(z_sums, acc_sums)

    plsc.subcore_barrier()

    for j in range(NCHUNK):
        copies[j].wait()
        pltpu.sync_copy(rows_v.at[j], acc_sums.at[keys_v.at[j]], add=True)

    plsc.subcore_barrier()

    @pl.when(sid == 0)
    def _():
        pltpu.sync_copy(acc_sums, sums_out.at[cid])


TC_BLOCK = 4096
TC_GRID = N_ROWS // TC_BLOCK
SC_BLOCKS = SC_ROWS // TC_BLOCK


def _tc_partial_body(mo_ref, tg_ref, ts_ref, out_ref, cnt_ref):
    i = pl.program_id(0)

    @pl.when(i == 0)
    def _():
        out_ref[...] = jnp.zeros((NUM_KEYS, D), jnp.float32)
        cnt_ref[...] = jnp.zeros((NUM_KEYS, D), jnp.float32)

    key = tg_ref[...] * NUM_SLICES + ts_ref[...]
    onehot_t = (key == lax.broadcasted_iota(jnp.int32, (NUM_KEYS, 1), 0))
    onehot_t = onehot_t.astype(jnp.float32)
    cnt_ref[...] += jnp.broadcast_to(
        jnp.sum(onehot_t, axis=1, keepdims=True), (NUM_KEYS, D))

    @pl.when(i >= SC_BLOCKS)
    def _():
        out_ref[...] += lax.dot_general(
            onehot_t, mo_ref[...], (((1,), (0,)), ((), ())),
            preferred_element_type=jnp.float32)


_tc_partial = pl.pallas_call(
    _tc_partial_body,
    grid=(TC_GRID,),
    in_specs=[
        pl.BlockSpec((TC_BLOCK, D), lambda i: (jnp.maximum(i, SC_BLOCKS), 0)),
        pl.BlockSpec((1, TC_BLOCK), lambda i: (0, i)),
        pl.BlockSpec((1, TC_BLOCK), lambda i: (0, i)),
    ],
    out_specs=(pl.BlockSpec((NUM_KEYS, D), lambda i: (0, 0)),
               pl.BlockSpec((NUM_KEYS, D), lambda i: (0, 0))),
    out_shape=(jax.ShapeDtypeStruct((NUM_KEYS, D), jnp.float32),
               jax.ShapeDtypeStruct((NUM_KEYS, D), jnp.float32)),
)


def _finalize_body(sc_ref, tc_ref, tc_cnt_ref, out_ref):
    s = sc_ref[...]
    s16 = s[0] + s[1] + tc_ref[...]
    cnt = tc_cnt_ref[...][:, 0:1]

    neg_cnt, pos_cnt = cnt[:NUM_SLICES], cnt[NUM_SLICES:]
    present = (neg_cnt + pos_cnt > 0).astype(jnp.float32)
    n = jnp.sum(present, axis=(0, 1), keepdims=True)

    inv_n = 1.0 / jnp.maximum(n, 1.0)
    inv_nm1 = 1.0 / jnp.maximum(n - 1.0, 1.0)

    def var_mean(sums, c):
        means = jnp.where(c > 0, sums / jnp.maximum(c, 1.0), 0.0)
        mu = jnp.sum(means * present, axis=0, keepdims=True) * inv_n
        var = jnp.sum(((means - mu) ** 2) * present, axis=0,
                      keepdims=True) * inv_nm1
        return jnp.mean(var, axis=1, keepdims=True)

    penalty = (var_mean(s16[:NUM_SLICES], neg_cnt)
               + var_mean(s16[NUM_SLICES:], pos_cnt)) / 2.0
    penalty = jnp.where(n < 2, 0.0, PENALTY_WEIGHT * penalty)
    out_ref[...] = penalty


_finalize = pl.pallas_call(
    _finalize_body,
    out_shape=jax.ShapeDtypeStruct((1, 1), jnp.float32),
)


def kernel(model_output, targets, time_slices):
    tg = targets.astype(jnp.int32)
    ts = time_slices.astype(jnp.int32)
    tg_row = tg.reshape(1, N_ROWS)
    ts_row = ts.reshape(1, N_ROWS)
    sc_sums = _segment_sums(model_output, tg, ts)
    tc_sums, tc_cnts = _tc_partial(model_output, tg_row, ts_row)
    out = _finalize(sc_sums, tc_sums, tc_cnts)
    return out[0, 0]
